# baseline (speedup 1.0000x reference)
import functools

import jax
import jax.numpy as jnp
from jax.experimental import pallas as pl
from jax.experimental.pallas import tpu as pltpu

K = 32
F_ITERS = 3


def _count_ge(zv, mid):
    m = (zv >= mid[None, :, :]).astype(jnp.float32)
    c1 = jnp.sum(m, axis=2)
    return jnp.sum(c1, axis=0)[:, None]


def _fused_kernel(x_ref, we_ref, be_ref, wd_ref, bd_ref,
                  xh_ref, zsp_ref, z_s, thr_s, bm_lo, bm_hi,
                  st_lo, st_hi, st_cl, acc,
                  *, nt, nd, dt, tb):
    i = pl.program_id(0)
    j = pl.program_id(1)
    cur = jax.lax.rem(i, 2)
    prev = jax.lax.rem(i + 1, 2)
    kf = jnp.float32(K)
    qpt = -(-K // nd)
    cw = dt // qpt

    zold = z_s[cur, j]
    thr2 = thr_s[cur]
    zsp = jnp.where(zold >= thr2, zold, 0.0)
    zsp_ref[...] = zsp
    dec = jax.lax.dot_general(zsp, wd_ref[...], (((1,), (1,)), ((), ())),
                              preferred_element_type=jnp.float32)
    acc[...] = jnp.where(j == 0, dec, acc[...] + dec)

    @pl.when(j == nd - 1)
    def _emit_xhat():
        xh_ref[...] = acc[...] + bd_ref[...]

    z = jax.lax.dot_general(x_ref[...], we_ref[...], (((1,), (1,)), ((), ())),
                            preferred_element_type=jnp.float32)
    z = z + be_ref[...]
    z_s[cur, j] = z
    cmn = jnp.max(z[:, 0:cw], axis=1, keepdims=True)
    cmx = cmn
    for q in range(1, qpt):
        m_q = jnp.max(z[:, q * cw:(q + 1) * cw], axis=1, keepdims=True)
        cmn = jnp.minimum(cmn, m_q)
        cmx = jnp.maximum(cmx, m_q)
    bm_lo[cur] = jnp.where(j == 0, cmn, jnp.minimum(bm_lo[cur], cmn))
    bm_hi[cur] = jnp.where(j == 0, cmx, jnp.maximum(bm_hi[cur], cmx))

    zv = z_s[prev]
    lo = st_lo[...]
    hi = st_hi[...]
    cl = st_cl[...]
    for _ in range(F_ITERS):
        mid = 0.5 * (lo + hi)
        cnt = _count_ge(zv, mid)
        ge = cnt >= kf
        lo = jnp.where(ge, mid, lo)
        hi = jnp.where(ge, hi, mid)
        cl = jnp.where(ge, cnt, cl)
    st_lo[...] = lo
    st_hi[...] = hi
    st_cl[...] = cl

    @pl.when(jnp.logical_and(i >= 1, jnp.logical_and(i <= nt, j == nd - 1)))
    def _finish():
        def cond(c):
            it, lo_, hi_, cl_ = c
            return jnp.logical_and(it < 40, jnp.any(cl_ != kf))

        def body(c):
            it, lo_, hi_, cl_ = c
            mid = 0.5 * (lo_ + hi_)
            cnt = _count_ge(zv, mid)
            ge = cnt >= kf
            return (it + 1,
                    jnp.where(ge, mid, lo_),
                    jnp.where(ge, hi_, mid),
                    jnp.where(ge, cnt, cl_))

        _, thr, _, _ = jax.lax.while_loop(
            cond, body, (jnp.int32(0), st_lo[...], st_hi[...], st_cl[...]))
        thr_s[prev] = thr

    @pl.when(j == nd - 1)
    def _seed():
        st_lo[...] = bm_lo[cur]
        st_hi[...] = bm_hi[cur]
        st_cl[...] = jnp.full((tb, 1), kf + 1.0, jnp.float32)


def kernel(x, W_enc, b_enc, W_dec, b_dec):
    n_tok, d_in = x.shape
    d_dict = W_enc.shape[0]
    tb = min(256, n_tok)
    dt = 1024
    nt = n_tok // tb
    nd = d_dict // dt
    b_enc2 = b_enc.reshape(1, d_dict)
    b_dec2 = b_dec.reshape(1, d_in)

    x_hat, z_sparse = pl.pallas_call(
        functools.partial(_fused_kernel, nt=nt, nd=nd, dt=dt, tb=tb),
        grid=(nt + 2, nd),
        in_specs=[
            pl.BlockSpec((tb, d_in),
                         lambda i, j: (jnp.minimum(i, nt - 1), 0)),
            pl.BlockSpec((dt, d_in), lambda i, j: (j, 0)),
            pl.BlockSpec((1, dt), lambda i, j: (0, j)),
            pl.BlockSpec((d_in, dt), lambda i, j: (0, j)),
            pl.BlockSpec((1, d_in), lambda i, j: (0, 0)),
        ],
        out_specs=[
            pl.BlockSpec((tb, d_in), lambda i, j: (jnp.maximum(i - 2, 0), 0)),
            pl.BlockSpec((tb, dt),
                         lambda i, j: (jnp.maximum(i - 2, 0), j)),
        ],
        out_shape=[
            jax.ShapeDtypeStruct((n_tok, d_in), jnp.float32),
            jax.ShapeDtypeStruct((n_tok, d_dict), jnp.float32),
        ],
        scratch_shapes=[
            pltpu.VMEM((2, nd, tb, dt), jnp.float32),
            pltpu.VMEM((2, tb, 1), jnp.float32),
            pltpu.VMEM((2, tb, 1), jnp.float32),
            pltpu.VMEM((2, tb, 1), jnp.float32),
            pltpu.VMEM((tb, 1), jnp.float32),
            pltpu.VMEM((tb, 1), jnp.float32),
            pltpu.VMEM((tb, 1), jnp.float32),
            pltpu.VMEM((tb, d_in), jnp.float32),
        ],
    )(x, W_enc, b_enc2, W_dec, b_dec2)

    return (x_hat, z_sparse)


# speedup vs baseline: 1.0956x; 1.0956x over previous
import functools

import jax
import jax.numpy as jnp
from jax.experimental import pallas as pl
from jax.experimental.pallas import tpu as pltpu

K = 32
F_ITERS = 1


def _count_ge(zv, mid):
    m = (zv >= mid[None, :, :]).astype(jnp.float32)
    c1 = jnp.sum(m, axis=2)
    return jnp.sum(c1, axis=0)[:, None]


def _fused_kernel(x_ref, we_ref, be_ref, wd_ref, bd_ref,
                  xh_ref, zsp_ref, z_s, thr_s, bm_lo, bm_hi,
                  st_lo, st_hi, st_cl, acc,
                  *, nt, nd, dt, tb):
    i = pl.program_id(0)
    j = pl.program_id(1)
    cur = jax.lax.rem(i, 2)
    prev = jax.lax.rem(i + 1, 2)
    kf = jnp.float32(K)
    qpt = -(-K // nd)
    cw = dt // qpt

    zold = z_s[cur, j]
    thr2 = thr_s[cur]
    zsp = jnp.where(zold >= thr2, zold, 0.0)
    zsp_ref[...] = zsp
    dec = jax.lax.dot_general(zsp, wd_ref[...], (((1,), (1,)), ((), ())),
                              preferred_element_type=jnp.float32)
    acc[...] = jnp.where(j == 0, dec, acc[...] + dec)

    @pl.when(j == nd - 1)
    def _emit_xhat():
        xh_ref[...] = acc[...] + bd_ref[...]

    z = jax.lax.dot_general(x_ref[...], we_ref[...], (((1,), (1,)), ((), ())),
                            preferred_element_type=jnp.float32)
    z = z + be_ref[...]
    z_s[cur, j] = z
    cmn = jnp.max(z[:, 0:cw], axis=1, keepdims=True)
    cmx = cmn
    for q in range(1, qpt):
        m_q = jnp.max(z[:, q * cw:(q + 1) * cw], axis=1, keepdims=True)
        cmn = jnp.minimum(cmn, m_q)
        cmx = jnp.maximum(cmx, m_q)
    bm_lo[cur] = jnp.where(j == 0, cmn, jnp.minimum(bm_lo[cur], cmn))
    bm_hi[cur] = jnp.where(j == 0, cmx, jnp.maximum(bm_hi[cur], cmx))

    zv = z_s[prev]
    lo = st_lo[...]
    hi = st_hi[...]
    cl = st_cl[...]
    for _ in range(F_ITERS):
        mid = 0.5 * (lo + hi)
        cnt = _count_ge(zv, mid)
        ge = cnt >= kf
        lo = jnp.where(ge, mid, lo)
        hi = jnp.where(ge, hi, mid)
        cl = jnp.where(ge, cnt, cl)
    st_lo[...] = lo
    st_hi[...] = hi
    st_cl[...] = cl

    @pl.when(jnp.logical_and(i >= 1, jnp.logical_and(i <= nt, j == nd - 1)))
    def _finish():
        def cond(c):
            it, lo_, hi_, cl_ = c
            return jnp.logical_and(it < 40, jnp.any(cl_ != kf))

        def body(c):
            it, lo_, hi_, cl_ = c
            mid = 0.5 * (lo_ + hi_)
            cnt = _count_ge(zv, mid)
            ge = cnt >= kf
            return (it + 1,
                    jnp.where(ge, mid, lo_),
                    jnp.where(ge, hi_, mid),
                    jnp.where(ge, cnt, cl_))

        _, thr, _, _ = jax.lax.while_loop(
            cond, body, (jnp.int32(0), st_lo[...], st_hi[...], st_cl[...]))
        thr_s[prev] = thr

    @pl.when(j == nd - 1)
    def _seed():
        st_lo[...] = bm_lo[cur]
        st_hi[...] = bm_hi[cur]
        st_cl[...] = jnp.full((tb, 1), kf + 1.0, jnp.float32)


def kernel(x, W_enc, b_enc, W_dec, b_dec):
    n_tok, d_in = x.shape
    d_dict = W_enc.shape[0]
    tb = min(256, n_tok)
    dt = 1024
    nt = n_tok // tb
    nd = d_dict // dt
    b_enc2 = b_enc.reshape(1, d_dict)
    b_dec2 = b_dec.reshape(1, d_in)

    x_hat, z_sparse = pl.pallas_call(
        functools.partial(_fused_kernel, nt=nt, nd=nd, dt=dt, tb=tb),
        grid=(nt + 2, nd),
        in_specs=[
            pl.BlockSpec((tb, d_in),
                         lambda i, j: (jnp.minimum(i, nt - 1), 0)),
            pl.BlockSpec((dt, d_in), lambda i, j: (j, 0)),
            pl.BlockSpec((1, dt), lambda i, j: (0, j)),
            pl.BlockSpec((d_in, dt), lambda i, j: (0, j)),
            pl.BlockSpec((1, d_in), lambda i, j: (0, 0)),
        ],
        out_specs=[
            pl.BlockSpec((tb, d_in), lambda i, j: (jnp.maximum(i - 2, 0), 0)),
            pl.BlockSpec((tb, dt),
                         lambda i, j: (jnp.maximum(i - 2, 0), j)),
        ],
        out_shape=[
            jax.ShapeDtypeStruct((n_tok, d_in), jnp.float32),
            jax.ShapeDtypeStruct((n_tok, d_dict), jnp.float32),
        ],
        scratch_shapes=[
            pltpu.VMEM((2, nd, tb, dt), jnp.float32),
            pltpu.VMEM((2, tb, 1), jnp.float32),
            pltpu.VMEM((2, tb, 1), jnp.float32),
            pltpu.VMEM((2, tb, 1), jnp.float32),
            pltpu.VMEM((tb, 1), jnp.float32),
            pltpu.VMEM((tb, 1), jnp.float32),
            pltpu.VMEM((tb, 1), jnp.float32),
            pltpu.VMEM((tb, d_in), jnp.float32),
        ],
    )(x, W_enc, b_enc2, W_dec, b_dec2)

    return (x_hat, z_sparse)


# speedup vs baseline: 1.2108x; 1.1052x over previous
import functools

import jax
import jax.numpy as jnp
from jax.experimental import pallas as pl
from jax.experimental.pallas import tpu as pltpu

K = 32
F_ITERS = 2


def _count_ge(zv, mid):
    m = (zv >= mid[None, :, :]).astype(jnp.float32)
    c1 = jnp.sum(m, axis=2)
    return jnp.sum(c1, axis=0)[:, None]


def _fused_kernel(x_ref, we_ref, be_ref, wd_ref, bd_ref,
                  xh_ref, zsp_ref, z_s, thr_s, bm_lo, bm_hi,
                  st_lo, st_hi, st_cl, acc,
                  *, nt, nd, dt, tb):
    i = pl.program_id(0)
    j = pl.program_id(1)
    cur = jax.lax.rem(i, 2)
    prev = jax.lax.rem(i + 1, 2)
    kf = jnp.float32(K)
    qpt = -(-K // nd)
    cw = dt // qpt

    zold = z_s[cur, j]
    thr2 = thr_s[cur]
    zsp = jnp.where(zold >= thr2, zold, 0.0)
    zsp_ref[...] = zsp
    dec = jax.lax.dot_general(zsp, wd_ref[...], (((1,), (1,)), ((), ())),
                              preferred_element_type=jnp.float32)
    acc[...] = jnp.where(j == 0, dec, acc[...] + dec)

    @pl.when(j == nd - 1)
    def _emit_xhat():
        xh_ref[...] = acc[...] + bd_ref[...]

    z = jax.lax.dot_general(x_ref[...], we_ref[...], (((1,), (1,)), ((), ())),
                            preferred_element_type=jnp.float32)
    z = z + be_ref[...]
    z_s[cur, j] = z
    cmn = jnp.max(z[:, 0:cw], axis=1, keepdims=True)
    cmx = cmn
    for q in range(1, qpt):
        m_q = jnp.max(z[:, q * cw:(q + 1) * cw], axis=1, keepdims=True)
        cmn = jnp.minimum(cmn, m_q)
        cmx = jnp.maximum(cmx, m_q)
    bm_lo[cur] = jnp.where(j == 0, cmn, jnp.minimum(bm_lo[cur], cmn))
    bm_hi[cur] = jnp.where(j == 0, cmx, jnp.maximum(bm_hi[cur], cmx))

    zv = z_s[prev]
    lo = st_lo[...]
    hi = st_hi[...]
    cl = st_cl[...]
    for _ in range(F_ITERS):
        mid = 0.5 * (lo + hi)
        cnt = _count_ge(zv, mid)
        ge = cnt >= kf
        lo = jnp.where(ge, mid, lo)
        hi = jnp.where(ge, hi, mid)
        cl = jnp.where(ge, cnt, cl)
    st_lo[...] = lo
    st_hi[...] = hi
    st_cl[...] = cl

    @pl.when(jnp.logical_and(i >= 1, jnp.logical_and(i <= nt, j == nd - 1)))
    def _finish():
        def cond(c):
            it, lo_, hi_, cl_ = c
            return jnp.logical_and(it < 40, jnp.any(cl_ != kf))

        def body(c):
            it, lo_, hi_, cl_ = c
            mid = 0.5 * (lo_ + hi_)
            cnt = _count_ge(zv, mid)
            ge = cnt >= kf
            return (it + 1,
                    jnp.where(ge, mid, lo_),
                    jnp.where(ge, hi_, mid),
                    jnp.where(ge, cnt, cl_))

        _, thr, _, _ = jax.lax.while_loop(
            cond, body, (jnp.int32(0), st_lo[...], st_hi[...], st_cl[...]))
        thr_s[prev] = thr

    @pl.when(j == nd - 1)
    def _seed():
        st_lo[...] = bm_lo[cur]
        st_hi[...] = bm_hi[cur]
        st_cl[...] = jnp.full((tb, 1), kf + 1.0, jnp.float32)


def kernel(x, W_enc, b_enc, W_dec, b_dec):
    n_tok, d_in = x.shape
    d_dict = W_enc.shape[0]
    tb = min(256, n_tok)
    dt = 1024
    nt = n_tok // tb
    nd = d_dict // dt
    b_enc2 = b_enc.reshape(1, d_dict)
    b_dec2 = b_dec.reshape(1, d_in)

    x_hat, z_sparse = pl.pallas_call(
        functools.partial(_fused_kernel, nt=nt, nd=nd, dt=dt, tb=tb),
        grid=(nt + 2, nd),
        in_specs=[
            pl.BlockSpec((tb, d_in),
                         lambda i, j: (jnp.minimum(i, nt - 1), 0)),
            pl.BlockSpec((dt, d_in), lambda i, j: (j, 0)),
            pl.BlockSpec((1, dt), lambda i, j: (0, j)),
            pl.BlockSpec((d_in, dt), lambda i, j: (0, j)),
            pl.BlockSpec((1, d_in), lambda i, j: (0, 0)),
        ],
        out_specs=[
            pl.BlockSpec((tb, d_in), lambda i, j: (jnp.maximum(i - 2, 0), 0)),
            pl.BlockSpec((tb, dt),
                         lambda i, j: (jnp.maximum(i - 2, 0), j)),
        ],
        out_shape=[
            jax.ShapeDtypeStruct((n_tok, d_in), jnp.float32),
            jax.ShapeDtypeStruct((n_tok, d_dict), jnp.float32),
        ],
        scratch_shapes=[
            pltpu.VMEM((2, nd, tb, dt), jnp.float32),
            pltpu.VMEM((2, tb, 1), jnp.float32),
            pltpu.VMEM((2, tb, 1), jnp.float32),
            pltpu.VMEM((2, tb, 1), jnp.float32),
            pltpu.VMEM((tb, 1), jnp.float32),
            pltpu.VMEM((tb, 1), jnp.float32),
            pltpu.VMEM((tb, 1), jnp.float32),
            pltpu.VMEM((tb, d_in), jnp.float32),
        ],
    )(x, W_enc, b_enc2, W_dec, b_dec2)

    return (x_hat, z_sparse)
